# Initial kernel scaffold; baseline (speedup 1.0000x reference)
import jax
import jax.numpy as jnp
from jax.experimental import pallas as pl


def kernel(x, w_gate, b_gate, w_noise, b_noise):
    raise NotImplementedError("write your pallas kernel here")


# speedup vs baseline: 5.1995x; 5.1995x over previous
import functools

import jax
import jax.numpy as jnp
from jax.experimental import pallas as pl
from jax.experimental.pallas import tpu as pltpu

_TOP_K = 8
_BLOCK_N = 256


def _gating_block_kernel(x_ref, w_ref, b_ref, gates_ref, load_ref, *, k_top):
    x = x_ref[...]
    w = w_ref[...]
    e = w.shape[0]
    bn = x.shape[0]
    logits = jax.lax.dot_general(
        w, x, (((1,), (1,)), ((), ())), preferred_element_type=jnp.float32)
    logits = logits + b_ref[...].reshape(e, 1)

    row = jax.lax.broadcasted_iota(jnp.int32, (e, bn), 0)
    work = logits
    acc = jnp.zeros((e, bn), jnp.float32)
    denom = jnp.zeros((1, bn), jnp.float32)
    m0 = jnp.max(work, axis=0, keepdims=True)
    for k in range(k_top):
        m = m0 if k == 0 else jnp.max(work, axis=0, keepdims=True)
        is_max = work == m
        sel = jnp.min(jnp.where(is_max, row, e), axis=0, keepdims=True)
        onehot = row == sel
        ex = jnp.exp(m - m0)
        acc = acc + jnp.where(onehot, ex, jnp.float32(0.0))
        denom = denom + ex
        work = jnp.where(onehot, -jnp.inf, work)

    gates_t = acc / denom
    gates = gates_t.T
    gates_ref[...] = gates

    counts = jnp.sum((gates > 0).astype(jnp.int32), axis=0, keepdims=True)

    @pl.when(pl.program_id(1) == 0)
    def _init():
        load_ref[...] = counts[None]

    @pl.when(pl.program_id(1) != 0)
    def _accumulate():
        load_ref[...] += counts[None]


def kernel(x, w_gate, b_gate, w_noise, b_noise):
    del w_noise, b_noise
    n, d = x.shape
    e = w_gate.shape[0]
    bn = min(_BLOCK_N, n)
    nb = n // bn
    cores = 2 if nb % 2 == 0 else 1
    half = nb // cores
    b2 = b_gate.reshape(1, e)

    gates, load3 = pl.pallas_call(
        functools.partial(_gating_block_kernel, k_top=_TOP_K),
        grid=(cores, half),
        in_specs=[
            pl.BlockSpec((bn, d), lambda i, j: (i * half + j, 0)),
            pl.BlockSpec((e, d), lambda i, j: (0, 0)),
            pl.BlockSpec((1, e), lambda i, j: (0, 0)),
        ],
        out_specs=[
            pl.BlockSpec((bn, e), lambda i, j: (i * half + j, 0)),
            pl.BlockSpec((1, 1, e), lambda i, j: (i, 0, 0)),
        ],
        out_shape=[
            jax.ShapeDtypeStruct((n, e), x.dtype),
            jax.ShapeDtypeStruct((cores, 1, e), jnp.int32),
        ],
        compiler_params=pltpu.CompilerParams(
            dimension_semantics=("parallel", "arbitrary")),
    )(x, w_gate, b2)

    load = load3.sum(axis=(0, 1))
    return gates, load
